# baseline (speedup 1.0000x reference)
import functools

import numpy as np
import jax
import jax.numpy as jnp
from jax import lax
from jax.experimental import pallas as pl
from jax.experimental.pallas import tpu as pltpu
from jax.experimental.pallas import tpu_sc as plsc

_VOCAB = 1000000
_D = 64
_B = 1024
_S = 200
_NW = 32
_ROWS = _B * _S
_RPW = _ROWS // _NW
_CHUNK = 400
_NCHUNK = _RPW // _CHUNK
_REPS = _CHUNK // _S
_LANES = 16


def _pos_encoding() -> jnp.ndarray:
    pos = np.arange(_S, dtype=np.float32)[:, None]
    i = np.arange(_D // 2, dtype=np.float32)[None, :]
    angles = pos / np.power(10000.0, (2.0 * i) / _D)
    pe = np.zeros((_S, _D), dtype=np.float32)
    pe[:, 0::2] = np.sin(angles)
    pe[:, 1::2] = np.cos(angles)
    return jnp.asarray(pe)


_MESH = plsc.VectorSubcoreMesh(core_axis_name="c", subcore_axis_name="s")


@functools.partial(
    pl.kernel,
    out_type=jax.ShapeDtypeStruct((_ROWS, _D), jnp.float32),
    mesh=_MESH,
    compiler_params=pltpu.CompilerParams(use_tc_tiling_on_sc=False),
    scratch_types=[
        pltpu.VMEM((_RPW,), jnp.int32),
        pltpu.VMEM((_S, _D), jnp.float32),
        pltpu.VMEM((_CHUNK, _D), jnp.float32),
        pltpu.SemaphoreType.DMA,
    ],
)
def _emb_lookup(x_hbm, table_hbm, pe_hbm, out_hbm, idx_v, pe_v, buf_v, gsem):
    wid = lax.axis_index("s") * 2 + lax.axis_index("c")
    base = wid * _RPW
    pltpu.sync_copy(x_hbm.at[pl.ds(base, _RPW)], idx_v)
    pltpu.sync_copy(pe_hbm, pe_v)

    def chunk_body(g, carry):
        start = pl.multiple_of(g * _CHUNK, 8)
        pltpu.async_copy(
            table_hbm.at[idx_v.at[pl.ds(start, _CHUNK)]], buf_v, gsem
        ).wait()

        def add_body(i, c):
            for rep in range(_REPS):
                row = rep * _S + i
                for k in range(_D // _LANES):
                    sl = pl.ds(k * _LANES, _LANES)
                    plsc.addupdate(buf_v.at[row, sl], pe_v[i, sl])
            return c

        lax.fori_loop(0, _S, add_body, 0, unroll=2)

        pltpu.sync_copy(buf_v, out_hbm.at[pl.ds(base + start, _CHUNK)])
        return carry

    lax.fori_loop(0, _NCHUNK, chunk_body, 0)


def kernel(x, table):
    pe = _pos_encoding()
    out = _emb_lookup(x.reshape(-1), table, pe)
    return out.reshape(_B, _S, _D)
